# baseline (device time: 29656 ns/iter reference)
import jax
import jax.numpy as jnp
from jax import lax
from jax.experimental import pallas as pl
from jax.experimental.pallas import tpu as pltpu

N_DEV = 4
B_LOC = 2
SQ = 256
SKV = 256
HQ = 16
HQ_LOC = 4
DH = 64
D = 512
DHID = 256
BLK = 64


def _body(x_ref, wq_ref, k_ref, v_ref, wo_ref, out_ref,
          cwq, cwo, ctx_ref, swq, rwq, swo, rwo):
    my = lax.axis_index("i")

    barrier = pltpu.get_barrier_semaphore()
    for d in range(1, N_DEV):
        peer = lax.rem(my + d, N_DEV)
        pl.semaphore_signal(barrier, inc=1, device_id=(peer,),
                            device_id_type=pl.DeviceIdType.MESH)
    pl.semaphore_wait(barrier, N_DEV - 1)

    cwq[0] = wq_ref[...]
    cwo[0] = wo_ref[...]

    sends = []
    for d in range(1, N_DEV):
        peer = lax.rem(my + d, N_DEV)
        s = N_DEV - d
        r_wq = pltpu.make_async_remote_copy(
            src_ref=cwq.at[0], dst_ref=cwq.at[s],
            send_sem=swq.at[d - 1], recv_sem=rwq.at[s],
            device_id=(peer,), device_id_type=pl.DeviceIdType.MESH)
        r_wo = pltpu.make_async_remote_copy(
            src_ref=cwo.at[0], dst_ref=cwo.at[s],
            send_sem=swo.at[d - 1], recv_sem=rwo.at[s],
            device_id=(peer,), device_id_type=pl.DeviceIdType.MESH)
        r_wq.start()
        r_wo.start()
        sends.append((r_wq, r_wo))

    qb = lax.broadcasted_iota(jnp.int32, (SQ, SKV), 0) // BLK
    kb = lax.broadcasted_iota(jnp.int32, (SQ, SKV), 1) // BLK
    mask = (qb == kb) | (kb == 0) | (lax.rem(qb + kb, 3) == 0)

    for s in (0, 3, 1, 2):
        if s != 0:
            pltpu.make_async_remote_copy(
                src_ref=cwq.at[s], dst_ref=cwq.at[s],
                send_sem=swq.at[0], recv_sem=rwq.at[s],
                device_id=(my,), device_id_type=pl.DeviceIdType.MESH,
            ).wait_recv()
            pltpu.make_async_remote_copy(
                src_ref=cwo.at[s], dst_ref=cwo.at[s],
                send_sem=swo.at[0], recv_sem=rwo.at[s],
                device_id=(my,), device_id_type=pl.DeviceIdType.MESH,
            ).wait_recv()
        g = lax.rem(my + s, N_DEV)
        q2 = jnp.dot(x_ref[...], cwq[s], preferred_element_type=jnp.float32)
        q2 = q2.astype(jnp.bfloat16)
        for b in range(B_LOC):
            for hl in range(HQ_LOC):
                idx = b * HQ + g * HQ_LOC + hl
                kk = k_ref[idx]
                vv = v_ref[idx]
                qq = q2[b * SQ:(b + 1) * SQ, hl * DH:(hl + 1) * DH]
                sc = lax.dot_general(qq, kk, (((1,), (1,)), ((), ())),
                                     preferred_element_type=jnp.float32)
                sc = jnp.where(mask, sc * 0.125, jnp.float32(-1e9))
                m = jnp.max(sc, axis=1, keepdims=True)
                w = jnp.exp(sc - m)
                w = w / jnp.sum(w, axis=1, keepdims=True)
                cx = jnp.dot(w.astype(jnp.bfloat16), vv,
                             preferred_element_type=jnp.float32)
                ctx_ref[b * SQ:(b + 1) * SQ,
                        hl * DH:(hl + 1) * DH] = cx.astype(jnp.bfloat16)
        contrib = jnp.dot(ctx_ref[...], cwo[s],
                          preferred_element_type=jnp.float32)
        if s == 0:
            out_ref[...] = contrib
        else:
            out_ref[...] = out_ref[...] + contrib

    for r_wq, r_wo in sends:
        r_wq.wait_send()
        r_wo.wait_send()


def kernel(x, Wq, K_ext, V_ext, Wo):
    my = lax.axis_index("i")
    xb = x.reshape(B_LOC * SQ, D).astype(jnp.bfloat16)
    wq = Wq.astype(jnp.bfloat16)
    wo = Wo.astype(jnp.bfloat16)
    kb = lax.dynamic_slice_in_dim(K_ext, my * B_LOC, B_LOC, axis=0)
    vb = lax.dynamic_slice_in_dim(V_ext, my * B_LOC, B_LOC, axis=0)
    kb = jnp.transpose(kb, (0, 2, 1, 3)).reshape(
        B_LOC * HQ, SKV, DH).astype(jnp.bfloat16)
    vb = jnp.transpose(vb, (0, 2, 1, 3)).reshape(
        B_LOC * HQ, SKV, DH).astype(jnp.bfloat16)

    out = pl.pallas_call(
        _body,
        out_shape=jax.ShapeDtypeStruct((B_LOC * SQ, D), jnp.float32),
        in_specs=[pl.BlockSpec(memory_space=pltpu.VMEM)] * 5,
        out_specs=pl.BlockSpec(memory_space=pltpu.VMEM),
        scratch_shapes=[
            pltpu.VMEM((N_DEV, D, DHID), jnp.bfloat16),
            pltpu.VMEM((N_DEV, DHID, D), jnp.bfloat16),
            pltpu.VMEM((B_LOC * SQ, DHID), jnp.bfloat16),
            pltpu.SemaphoreType.DMA((N_DEV - 1,)),
            pltpu.SemaphoreType.DMA((N_DEV,)),
            pltpu.SemaphoreType.DMA((N_DEV - 1,)),
            pltpu.SemaphoreType.DMA((N_DEV,)),
        ],
        compiler_params=pltpu.CompilerParams(collective_id=0),
    )(xb, wq, kb, vb, wo)
    return out.reshape(B_LOC, SQ, D)
